# baseline (device time: 32043 ns/iter reference)
import jax
import jax.numpy as jnp
from jax import lax
from jax.experimental import pallas as pl
from jax.experimental.pallas import tpu as pltpu

N_DEV = 8
N_TOK = 512
D_IN = 256
D_OUT = 512
N_EXPERTS = 16
E_LOCAL = 2
ROWS = N_TOK // N_DEV


def kernel(x, router_W, route_idx, expert_W, shared_W):
    def body(x_ref, rw_ref, idx_ref, ew_ref, sw_ref, out_ref,
             partial_ref, comm_ref, send_sems, recv_sems):
        my = lax.axis_index("i")
        left = lax.rem(my - 1 + N_DEV, N_DEV)
        right = lax.rem(my + 1, N_DEV)

        barrier_sem = pltpu.get_barrier_semaphore()
        for nbr in (left, right):
            pl.semaphore_signal(
                barrier_sem, inc=1,
                device_id=(nbr,), device_id_type=pl.DeviceIdType.MESH,
            )
        pl.semaphore_wait(barrier_sem, 2)

        xf = x_ref[:, :]
        scores = jnp.dot(xf, rw_ref[:, :], preferred_element_type=jnp.float32)
        s_max = jnp.max(scores, axis=-1, keepdims=True)
        e = jnp.exp(scores - s_max)
        probs = e / jnp.sum(e, axis=-1, keepdims=True)

        idx = idx_ref[:, :]
        cols = lax.broadcasted_iota(jnp.int32, (N_TOK, N_EXPERTS), 1)
        p = jnp.sum(jnp.where(cols == idx, probs, 0.0), axis=-1,
                    keepdims=True)

        xb = xf.astype(jnp.bfloat16)
        e0 = my * E_LOCAL
        coeff0 = jnp.where(idx == e0, p, 0.0)
        coeff1 = jnp.where(idx == e0 + 1, p, 0.0)
        y0 = jnp.dot(xb, ew_ref[0, :, :].astype(jnp.bfloat16),
                     preferred_element_type=jnp.float32)
        y1 = jnp.dot(xb, ew_ref[1, :, :].astype(jnp.bfloat16),
                     preferred_element_type=jnp.float32)
        partial_ref[:, :] = coeff0 * y0 + coeff1 * y1

        x_mine = x_ref[pl.ds(my * ROWS, ROWS), :].astype(jnp.bfloat16)
        shared_mine = jnp.dot(x_mine, sw_ref[:, :].astype(jnp.bfloat16),
                              preferred_element_type=jnp.float32)

        comm_ref[0, :, :] = partial_ref[pl.ds(left * ROWS, ROWS), :]
        for h in range(N_DEV - 1):
            rdma = pltpu.make_async_remote_copy(
                src_ref=comm_ref.at[h],
                dst_ref=comm_ref.at[h + 1],
                send_sem=send_sems.at[h],
                recv_sem=recv_sems.at[h],
                device_id=(right,),
                device_id_type=pl.DeviceIdType.MESH,
            )
            rdma.start()
            rdma.wait()
            c = lax.rem(my - 2 - h + 2 * N_DEV, N_DEV)
            if h < N_DEV - 2:
                comm_ref[h + 1, :, :] += partial_ref[pl.ds(c * ROWS, ROWS), :]

        out_ref[:, :] = (comm_ref[N_DEV - 1, :, :]
                         + partial_ref[pl.ds(my * ROWS, ROWS), :]
                         + shared_mine)

    return pl.pallas_call(
        body,
        out_shape=jax.ShapeDtypeStruct((ROWS, D_OUT), jnp.float32),
        in_specs=[pl.BlockSpec(memory_space=pltpu.VMEM)] * 5,
        out_specs=pl.BlockSpec(memory_space=pltpu.VMEM),
        scratch_shapes=[
            pltpu.VMEM((N_TOK, D_OUT), jnp.float32),
            pltpu.VMEM((N_DEV, ROWS, D_OUT), jnp.float32),
            pltpu.SemaphoreType.DMA((N_DEV - 1,)),
            pltpu.SemaphoreType.DMA((N_DEV - 1,)),
        ],
        compiler_params=pltpu.CompilerParams(collective_id=0),
    )(x, router_W, route_idx, expert_W, shared_W)


# device time: 13837 ns/iter; 2.3157x vs baseline; 2.3157x over previous
import jax
import jax.numpy as jnp
from jax import lax
from jax.experimental import pallas as pl
from jax.experimental.pallas import tpu as pltpu

N_DEV = 8
N_TOK = 512
D_IN = 256
D_OUT = 512
N_EXPERTS = 16
E_LOCAL = 2
ROWS = N_TOK // N_DEV


def kernel(x, router_W, route_idx, expert_W, shared_W):
    def body(x_ref, rw_ref, idx_ref, ew_ref, sw_ref, out_ref,
             partial_ref, comm_ref, send_sems, recv_sems):
        my = lax.axis_index("i")

        barrier_sem = pltpu.get_barrier_semaphore()
        for j in range(1, N_DEV):
            peer = lax.rem(my + j, N_DEV)
            pl.semaphore_signal(
                barrier_sem, inc=1,
                device_id=(peer,), device_id_type=pl.DeviceIdType.MESH,
            )
        pl.semaphore_wait(barrier_sem, N_DEV - 1)

        xf = x_ref[:, :]
        scores = jnp.dot(xf, rw_ref[:, :], preferred_element_type=jnp.float32)
        s_max = jnp.max(scores, axis=-1, keepdims=True)
        e = jnp.exp(scores - s_max)
        probs = e / jnp.sum(e, axis=-1, keepdims=True)

        idx = idx_ref[:, :]
        cols = lax.broadcasted_iota(jnp.int32, (N_TOK, N_EXPERTS), 1)
        p = jnp.sum(jnp.where(cols == idx, probs, 0.0), axis=-1,
                    keepdims=True)

        xb = xf.astype(jnp.bfloat16)
        e0 = my * E_LOCAL
        coeff0 = jnp.where(idx == e0, p, 0.0)
        coeff1 = jnp.where(idx == e0 + 1, p, 0.0)
        y0 = jnp.dot(xb, ew_ref[0, :, :].astype(jnp.bfloat16),
                     preferred_element_type=jnp.float32)
        y1 = jnp.dot(xb, ew_ref[1, :, :].astype(jnp.bfloat16),
                     preferred_element_type=jnp.float32)
        partial_ref[:, :] = (coeff0 * y0 + coeff1 * y1).astype(jnp.bfloat16)

        rdmas = []
        for j in range(1, N_DEV):
            t = lax.rem(my + j, N_DEV)
            rdma = pltpu.make_async_remote_copy(
                src_ref=partial_ref.at[pl.ds(t * ROWS, ROWS), :],
                dst_ref=comm_ref.at[j - 1],
                send_sem=send_sems.at[j - 1],
                recv_sem=recv_sems.at[j - 1],
                device_id=(t,),
                device_id_type=pl.DeviceIdType.MESH,
            )
            rdma.start()
            rdmas.append(rdma)

        x_mine = x_ref[pl.ds(my * ROWS, ROWS), :].astype(jnp.bfloat16)
        shared_mine = jnp.dot(x_mine, sw_ref[:, :].astype(jnp.bfloat16),
                              preferred_element_type=jnp.float32)
        acc = shared_mine + partial_ref[pl.ds(my * ROWS, ROWS), :].astype(
            jnp.float32)

        for rdma in rdmas:
            rdma.wait()
        out_ref[:, :] = acc + jnp.sum(
            comm_ref[:, :, :].astype(jnp.float32), axis=0)

    return pl.pallas_call(
        body,
        out_shape=jax.ShapeDtypeStruct((ROWS, D_OUT), jnp.float32),
        in_specs=[pl.BlockSpec(memory_space=pltpu.VMEM)] * 5,
        out_specs=pl.BlockSpec(memory_space=pltpu.VMEM),
        scratch_shapes=[
            pltpu.VMEM((N_TOK, D_OUT), jnp.bfloat16),
            pltpu.VMEM((N_DEV - 1, ROWS, D_OUT), jnp.bfloat16),
            pltpu.SemaphoreType.DMA((N_DEV - 1,)),
            pltpu.SemaphoreType.DMA((N_DEV - 1,)),
        ],
        compiler_params=pltpu.CompilerParams(collective_id=0),
    )(x, router_W, route_idx, expert_W, shared_W)
